# baseline (speedup 1.0000x reference)
import functools

import jax
import jax.numpy as jnp
from jax import lax
from jax.experimental import pallas as pl
from jax.experimental.pallas import tpu as pltpu
from jax.experimental.pallas import tpu_sc as plsc

_N = 100000
_D = 64
_B = 4096
_LABEL = 0.95

_ROWS2 = _N * _D // 128
_BLK = 2000
_GRID = _ROWS2 // _BLK


def _base_body(s_ref, t_ref, o_ref):
    @pl.when(pl.program_id(0) == 0)
    def _():
        o_ref[0, 0] = jnp.float32(0.0)

    o_ref[0, 0] += jnp.sum(jnp.abs(s_ref[...] - t_ref[...]))


def _base_sum(s2, t2):
    return pl.pallas_call(
        _base_body,
        grid=(_GRID,),
        in_specs=[pl.BlockSpec((_BLK, 128), lambda i: (i, 0))] * 2,
        out_specs=pl.BlockSpec(memory_space=pltpu.SMEM),
        out_shape=jax.ShapeDtypeStruct((1, 1), jnp.float32),
    )(s2, t2)


_NT = 16
_RPT = _B // _NT
_CH = 128
_NCH = _RPT // _CH


def _corr_body(logits_hbm, cls_hbm, s_hbm, t_hbm, out_hbm,
               cls2, iot2, win2, coef2, srow, trow, lrow, accv, wsp, sem):
    wid = lax.axis_index("s")
    base = wid * _RPT

    for j in range(_NCH):
        off = base + j * _CH
        pltpu.sync_copy(cls_hbm.at[pl.ds(off, _CH)], cls2.at[j])
        pltpu.sync_copy(logits_hbm.at[pl.ds(off, _CH)], lrow.at[j])
        for k in range(_CH // 16):
            iot2[j, pl.ds(k * 16, 16)] = (
                lax.iota(jnp.int32, 16) + (off + k * 16))

    for j in range(_NCH):
        pltpu.sync_copy(iot2.at[j], wsp.at[cls2.at[j]])
    plsc.subcore_barrier()
    for j in range(_NCH):
        pltpu.sync_copy(wsp.at[cls2.at[j]], win2.at[j])
        pltpu.async_copy(s_hbm.at[cls2.at[j]], srow.at[j], sem).wait()
        pltpu.async_copy(t_hbm.at[cls2.at[j]], trow.at[j], sem).wait()

    for j in range(_NCH):
        for k in range(_CH // 16):
            sl = pl.ds(k * 16, 16)
            m = win2[j, sl] == iot2[j, sl]
            coef2[pl.ds(j * _CH + k * 16, 16)] = jnp.where(
                m, jnp.float32(1.0 - _LABEL), jnp.float32(0.0))

    def grp_body(g, acc):
        for j in range(_NCH):
            mgrp = coef2[pl.ds(j * _CH + g * 16, 16)]
            for l in range(16):
                mv = jnp.full((16,), mgrp[l], jnp.float32)
                r = g * 16 + l
                for k in range(_D // 16):
                    sl = pl.ds(k * 16, 16)
                    sv = srow[j, r, sl]
                    tv = trow[j, r, sl]
                    lv = lrow[j, r, sl]
                    nv = sv + mv * (lv - sv)
                    acc = acc + (jnp.abs(nv - tv) - jnp.abs(sv - tv))
        return acc

    acc = lax.fori_loop(0, _CH // 16, grp_body, jnp.zeros((16,), jnp.float32))
    accv[...] = acc
    pltpu.sync_copy(accv, out_hbm.at[wid])


def _corr_call(logits, the_class, s_logits, t_logits):
    mesh = plsc.VectorSubcoreMesh(
        core_axis_name="c", subcore_axis_name="s", num_cores=1)
    f = pl.kernel(
        _corr_body,
        out_type=jax.ShapeDtypeStruct((_NT, 16), jnp.float32),
        mesh=mesh,
        compiler_params=pltpu.CompilerParams(use_tc_tiling_on_sc=False),
        scratch_types=[
            pltpu.VMEM((_NCH, _CH), jnp.int32),
            pltpu.VMEM((_NCH, _CH), jnp.int32),
            pltpu.VMEM((_NCH, _CH), jnp.int32),
            pltpu.VMEM((_RPT,), jnp.float32),
            pltpu.VMEM((_NCH, _CH, _D), jnp.float32),
            pltpu.VMEM((_NCH, _CH, _D), jnp.float32),
            pltpu.VMEM((_NCH, _CH, _D), jnp.float32),
            pltpu.VMEM((16,), jnp.float32),
            pltpu.VMEM_SHARED((_N,), jnp.int32),
            pltpu.SemaphoreType.DMA,
        ],
    )
    return f(logits, the_class, s_logits, t_logits)


def kernel(logits, the_class, s_logits, t_logits):
    s2 = s_logits.reshape(_ROWS2, 128)
    t2 = t_logits.reshape(_ROWS2, 128)
    base = _base_sum(s2, t2)
    corr = _corr_call(logits, the_class, s_logits, t_logits)
    return (base[0, 0] + jnp.sum(corr)) / jnp.float32(_N)


# speedup vs baseline: 1.1359x; 1.1359x over previous
import functools

import jax
import jax.numpy as jnp
from jax import lax
from jax.experimental import pallas as pl
from jax.experimental.pallas import tpu as pltpu
from jax.experimental.pallas import tpu_sc as plsc

_N = 100000
_D = 64
_B = 4096
_LABEL = 0.95

_BLK = 4000
_GRID = _N // _BLK


def _base_body(s_ref, t_ref, o_ref):
    @pl.when(pl.program_id(0) == 0)
    def _():
        o_ref[0, 0] = jnp.float32(0.0)

    o_ref[0, 0] += jnp.sum(jnp.abs(s_ref[...] - t_ref[...]))


def _base_sum(s, t):
    return pl.pallas_call(
        _base_body,
        grid=(_GRID,),
        in_specs=[pl.BlockSpec((_BLK, _D), lambda i: (i, 0))] * 2,
        out_specs=pl.BlockSpec(memory_space=pltpu.SMEM),
        out_shape=jax.ShapeDtypeStruct((1, 1), jnp.float32),
    )(s, t)


_NT = 16
_RPT = _B // _NT
_CH = 128
_NCH = _RPT // _CH


def _corr_body(logits_hbm, cls_hbm, s_hbm, t_hbm, out_hbm,
               cls2, iot2, win2, coef2, srow, trow, lrow, accv, wsp, sem):
    wid = lax.axis_index("s")
    base = wid * _RPT

    for j in range(_NCH):
        off = base + j * _CH
        pltpu.sync_copy(cls_hbm.at[pl.ds(off, _CH)], cls2.at[j])
        pltpu.sync_copy(logits_hbm.at[pl.ds(off, _CH)], lrow.at[j])
        for k in range(_CH // 16):
            iot2[j, pl.ds(k * 16, 16)] = (
                lax.iota(jnp.int32, 16) + (off + k * 16))

    for j in range(_NCH):
        pltpu.sync_copy(iot2.at[j], wsp.at[cls2.at[j]])
    plsc.subcore_barrier()
    for j in range(_NCH):
        pltpu.sync_copy(wsp.at[cls2.at[j]], win2.at[j])
        pltpu.async_copy(s_hbm.at[cls2.at[j]], srow.at[j], sem).wait()
        pltpu.async_copy(t_hbm.at[cls2.at[j]], trow.at[j], sem).wait()

    for j in range(_NCH):
        for k in range(_CH // 16):
            sl = pl.ds(k * 16, 16)
            m = win2[j, sl] == iot2[j, sl]
            coef2[pl.ds(j * _CH + k * 16, 16)] = jnp.where(
                m, jnp.float32(1.0 - _LABEL), jnp.float32(0.0))

    def grp_body(g, acc):
        for j in range(_NCH):
            mgrp = coef2[pl.ds(j * _CH + g * 16, 16)]
            for l in range(16):
                mv = jnp.full((16,), mgrp[l], jnp.float32)
                r = g * 16 + l
                for k in range(_D // 16):
                    sl = pl.ds(k * 16, 16)
                    sv = srow[j, r, sl]
                    tv = trow[j, r, sl]
                    lv = lrow[j, r, sl]
                    nv = sv + mv * (lv - sv)
                    acc = acc + (jnp.abs(nv - tv) - jnp.abs(sv - tv))
        return acc

    acc = lax.fori_loop(0, _CH // 16, grp_body, jnp.zeros((16,), jnp.float32))
    accv[...] = acc
    pltpu.sync_copy(accv, out_hbm.at[wid])


def _corr_call(logits, the_class, s_logits, t_logits):
    mesh = plsc.VectorSubcoreMesh(
        core_axis_name="c", subcore_axis_name="s", num_cores=1)
    f = pl.kernel(
        _corr_body,
        out_type=jax.ShapeDtypeStruct((_NT, 16), jnp.float32),
        mesh=mesh,
        compiler_params=pltpu.CompilerParams(use_tc_tiling_on_sc=False),
        scratch_types=[
            pltpu.VMEM((_NCH, _CH), jnp.int32),
            pltpu.VMEM((_NCH, _CH), jnp.int32),
            pltpu.VMEM((_NCH, _CH), jnp.int32),
            pltpu.VMEM((_RPT,), jnp.float32),
            pltpu.VMEM((_NCH, _CH, _D), jnp.float32),
            pltpu.VMEM((_NCH, _CH, _D), jnp.float32),
            pltpu.VMEM((_NCH, _CH, _D), jnp.float32),
            pltpu.VMEM((16,), jnp.float32),
            pltpu.VMEM_SHARED((_N,), jnp.int32),
            pltpu.SemaphoreType.DMA,
        ],
    )
    return f(logits, the_class, s_logits, t_logits)


def kernel(logits, the_class, s_logits, t_logits):
    base = _base_sum(s_logits, t_logits)
    corr = _corr_call(logits, the_class, s_logits, t_logits)
    return (base[0, 0] + jnp.sum(corr)) / jnp.float32(_N)
